# Initial kernel scaffold; baseline (speedup 1.0000x reference)
import jax
import jax.numpy as jnp
from jax import lax
from jax.experimental import pallas as pl
from jax.experimental.pallas import tpu as pltpu
from jax.experimental.pallas import tpu_sc as plsc

N = 10000
E = 320000
D = 128
NP = 10240
RPT = NP // 16
EPT = E // 16
CE = 80
NCE = EPT // CE
CR = 64
NCR = RPT // CR
DH = D // 2


def _sc_body(x_hbm, src_hbm, dst_hbm, out_hbm, xs_hbm,
             deg_sh, acc_sh, zacc, zdeg, ones, dbuf, ibuf,
             xrow, xsbuf, sidx, didx, rows, abuf, sem):
    c = lax.axis_index("c")
    s = lax.axis_index("s")
    f0 = jnp.float32(0.0)

    def fill_zacc(i, _):
        for k in range(CR // 16):
            zacc[i, pl.ds(k * 16, 16)] = jnp.full((16,), f0)
        return _
    lax.fori_loop(0, CR, fill_zacc, None)

    def fill_zdeg(i, _):
        zdeg[pl.ds(i * 16, 16)] = jnp.full((16,), f0)
        return _
    lax.fori_loop(0, RPT // 16, fill_zdeg, None)

    def fill_ones(i, _):
        ones[pl.ds(i * 16, 16)] = jnp.full((16,), jnp.float32(1.0))
        return _
    lax.fori_loop(0, CE // 16, fill_ones, None)

    pltpu.sync_copy(zdeg, deg_sh.at[pl.ds(s * RPT, RPT)])

    def zero_acc(j, _):
        pltpu.sync_copy(zacc, acc_sh.at[pl.ds(s * RPT + j * CR, CR), :])
        return _
    lax.fori_loop(0, NCR, zero_acc, None)
    plsc.subcore_barrier()

    def deg_step(j, _):
        e0 = s * EPT + j * CE
        pltpu.sync_copy(src_hbm.at[pl.ds(e0, CE)], sidx)
        pltpu.sync_copy(ones, deg_sh.at[sidx], add=True)
        return _
    lax.fori_loop(0, NCE, deg_step, None)
    plsc.subcore_barrier()

    pltpu.sync_copy(deg_sh.at[pl.ds(s * RPT, RPT)], dbuf)

    def inv_step(i, _):
        d = jnp.maximum(dbuf[pl.ds(i * 16, 16)], jnp.float32(1.0))
        ii = plsc.bitcast(d, jnp.int32)
        ii = jnp.int32(0x5F3759DF) - (ii >> 1)
        y = plsc.bitcast(ii, jnp.float32)
        half = jnp.float32(0.5) * d
        y = y * (jnp.float32(1.5) - half * y * y)
        y = y * (jnp.float32(1.5) - half * y * y)
        y = y * (jnp.float32(1.5) - half * y * y)
        ibuf[pl.ds(i * 16, 16)] = y
        return _
    lax.fori_loop(0, RPT // 16, inv_step, None)

    def pre_chunk(j, _):
        r0 = s * RPT + j * CR
        pltpu.sync_copy(x_hbm.at[pl.ds(r0, CR), :], xrow)

        def pre_row(i, _):
            sc = ibuf[j * CR + i]
            for k in range(DH // 16):
                v = xrow[i, pl.ds(c * DH + k * 16, 16)]
                xsbuf[i, pl.ds(k * 16, 16)] = v * sc
            return _
        lax.fori_loop(0, CR, pre_row, None)
        pltpu.sync_copy(xsbuf, xs_hbm.at[pl.ds(c * NP + r0, CR), :])
        return _
    lax.fori_loop(0, NCR, pre_chunk, None)
    plsc.subcore_barrier()

    def edge_step(j, _):
        e0 = s * EPT + j * CE
        pltpu.sync_copy(src_hbm.at[pl.ds(e0, CE)], sidx)
        pltpu.sync_copy(dst_hbm.at[pl.ds(e0, CE)], didx)
        base = c * NP
        for k in range(CE // 16):
            sidx[pl.ds(k * 16, 16)] = sidx[pl.ds(k * 16, 16)] + base
        pltpu.async_copy(xs_hbm.at[sidx], rows, sem).wait()
        pltpu.sync_copy(rows, acc_sh.at[didx], add=True)
        return _
    lax.fori_loop(0, NCE, edge_step, None)
    plsc.subcore_barrier()

    def out_chunk(j, _):
        r0 = s * RPT + j * CR
        pltpu.sync_copy(acc_sh.at[pl.ds(r0, CR), :], abuf)

        def out_row(i, _):
            sc = ibuf[j * CR + i]
            for k in range(DH // 16):
                abuf[i, pl.ds(k * 16, 16)] = abuf[i, pl.ds(k * 16, 16)] * sc
            return _
        lax.fori_loop(0, CR, out_row, None)
        pltpu.sync_copy(abuf, out_hbm.at[pl.ds(r0, CR), pl.ds(c * DH, DH)])
        return _
    lax.fori_loop(0, NCR, out_chunk, None)


@jax.jit
def _gcn_sc(x_pad, src, dst):
    mesh = plsc.VectorSubcoreMesh(core_axis_name="c", subcore_axis_name="s")
    k = pl.kernel(
        _sc_body,
        out_type=(
            jax.ShapeDtypeStruct((NP, D), jnp.float32),
            jax.ShapeDtypeStruct((2 * NP, DH), jnp.float32),
        ),
        mesh=mesh,
        scratch_types=[
            pltpu.VMEM_SHARED((NP,), jnp.float32),
            pltpu.VMEM_SHARED((NP, DH), jnp.float32),
            pltpu.VMEM((CR, CR), jnp.float32),
            pltpu.VMEM((RPT,), jnp.float32),
            pltpu.VMEM((CE,), jnp.float32),
            pltpu.VMEM((RPT,), jnp.float32),
            pltpu.VMEM((RPT,), jnp.float32),
            pltpu.VMEM((CR, D), jnp.float32),
            pltpu.VMEM((CR, DH), jnp.float32),
            pltpu.VMEM((CE,), jnp.int32),
            pltpu.VMEM((CE,), jnp.int32),
            pltpu.VMEM((CE, DH), jnp.float32),
            pltpu.VMEM((CR, CR), jnp.float32),
            pltpu.SemaphoreType.DMA,
        ],
    )
    out, _ = k(x_pad, src, dst)
    return out


def kernel(x, edge_index):
    src = edge_index[0].astype(jnp.int32)
    dst = edge_index[1].astype(jnp.int32)
    x_pad = jnp.pad(x, ((0, NP - N), (0, 0)))
    out = _gcn_sc(x_pad, src, dst)
    return out[:N]


# speedup vs baseline: 13.3881x; 13.3881x over previous
import jax
import jax.numpy as jnp
from jax import lax
from jax.experimental import pallas as pl
from jax.experimental.pallas import tpu as pltpu
from jax.experimental.pallas import tpu_sc as plsc

N = 10000
E = 320000
D = 128
NP = 10240
RPT = NP // 16
HR = NP // 128
HRT = HR // 16
EPT = E // 16
CE = 80
NCD = EPT // CE
EPC = E // 2
EPCT = EPC // 16
NCE = EPCT // CE
CR = 64
NCR = RPT // CR


def _rsqrt16(d):
    ii = lax.bitcast_convert_type(d, jnp.int32)
    ii = jnp.int32(0x5F3759DF) - (ii >> 1)
    y = lax.bitcast_convert_type(ii, jnp.float32)
    half = jnp.float32(0.5) * d
    y = y * (jnp.float32(1.5) - half * y * y)
    y = y * (jnp.float32(1.5) - half * y * y)
    y = y * (jnp.float32(1.5) - half * y * y)
    return y


def _main_body(x_hbm, src_hbm, dst_hbm, part_hbm, xs_hbm,
               deg_sh, acc_sh, zdeg, iden, dbuf, ibuf,
               xrow, sidx, didx, rows, abuf, sem):
    hist = rows
    zacc = abuf
    c = lax.axis_index("c")
    s = lax.axis_index("s")
    f0 = jnp.float32(0.0)

    def fill_zacc(i, _):
        for k in range(D // 16):
            zacc[i, pl.ds(k * 16, 16)] = jnp.full((16,), f0)
        return _
    lax.fori_loop(0, CR, fill_zacc, None)

    def fill_zdeg(i, _):
        for k in range(D // 16):
            zdeg[i, pl.ds(k * 16, 16)] = jnp.full((16,), f0)
        return _
    lax.fori_loop(0, HRT, fill_zdeg, None)

    def fill_iden(k, _):
        iden[pl.ds(k * 16, 16)] = lax.iota(jnp.int32, 16) + k * 16
        return _
    lax.fori_loop(0, HR // 16, fill_iden, None)

    def fill_hist(i, _):
        for k in range(D // 16):
            hist[i, pl.ds(k * 16, 16)] = jnp.full((16,), f0)
        return _
    lax.fori_loop(0, HR, fill_hist, None)

    pltpu.sync_copy(zdeg, deg_sh.at[pl.ds(s * HRT, HRT), :])

    def zero_acc(j, _):
        pltpu.sync_copy(zacc, acc_sh.at[pl.ds(s * RPT + j * CR, CR), :])
        return _
    lax.fori_loop(0, NCR, zero_acc, None)
    plsc.subcore_barrier()

    one16 = jnp.full((16,), jnp.float32(1.0))

    def deg_step(j, _):
        e0 = s * EPT + j * CE
        pltpu.sync_copy(src_hbm.at[pl.ds(e0, CE)], sidx)
        for k in range(CE // 16):
            n = sidx[pl.ds(k * 16, 16)]
            plsc.addupdate_scatter(hist, [n >> 7, n & 127], one16)
        return _
    lax.fori_loop(0, NCD, deg_step, None)
    pltpu.sync_copy(hist, deg_sh.at[iden], add=True)
    plsc.subcore_barrier()

    pltpu.sync_copy(deg_sh.at[pl.ds(s * HRT, HRT), :], dbuf)

    def inv_step(i, _):
        r = i // 8
        k = i % 8
        d = jnp.maximum(dbuf[r, pl.ds(k * 16, 16)], jnp.float32(1.0))
        ibuf[pl.ds(i * 16, 16)] = _rsqrt16(d)
        return _
    lax.fori_loop(0, RPT // 16, inv_step, None)

    def pre_chunk(j, _):
        r0 = s * RPT + j * CR
        pltpu.sync_copy(x_hbm.at[pl.ds(r0, CR), :], xrow)

        def pre_row(i, _):
            idxv = jnp.zeros((16,), jnp.int32) + (j * CR + i)
            sc = plsc.load_gather(ibuf, [idxv])
            for k in range(D // 16):
                xrow[i, pl.ds(k * 16, 16)] = xrow[i, pl.ds(k * 16, 16)] * sc
            return _
        lax.fori_loop(0, CR, pre_row, None)
        pltpu.sync_copy(xrow, xs_hbm.at[pl.ds(c * NP + r0, CR), :])
        return _
    lax.fori_loop(0, NCR, pre_chunk, None)
    plsc.subcore_barrier()

    def edge_step(j, _):
        e0 = c * EPC + s * EPCT + j * CE
        pltpu.sync_copy(src_hbm.at[pl.ds(e0, CE)], sidx)
        pltpu.sync_copy(dst_hbm.at[pl.ds(e0, CE)], didx)
        base = c * NP
        for k in range(CE // 16):
            sidx[pl.ds(k * 16, 16)] = sidx[pl.ds(k * 16, 16)] + base
        pltpu.async_copy(xs_hbm.at[sidx], rows, sem).wait()
        pltpu.sync_copy(rows, acc_sh.at[didx], add=True)
        return _
    lax.fori_loop(0, NCE, edge_step, None)
    plsc.subcore_barrier()

    def out_chunk(j, _):
        r0 = s * RPT + j * CR
        pltpu.sync_copy(acc_sh.at[pl.ds(r0, CR), :], abuf)

        def out_row(i, _):
            idxv = jnp.zeros((16,), jnp.int32) + (j * CR + i)
            sc = plsc.load_gather(ibuf, [idxv])
            for k in range(D // 16):
                abuf[i, pl.ds(k * 16, 16)] = abuf[i, pl.ds(k * 16, 16)] * sc
            return _
        lax.fori_loop(0, CR, out_row, None)
        pltpu.sync_copy(abuf, part_hbm.at[c, pl.ds(r0, CR), :])
        return _
    lax.fori_loop(0, NCR, out_chunk, None)


def _sum_body(part_hbm, out_hbm, p0, p1):
    c = lax.axis_index("c")
    s = lax.axis_index("s")
    wid = s * 2 + c
    rpw = NP // 32

    def chunk(j, _):
        r0 = wid * rpw + j * CR
        pltpu.sync_copy(part_hbm.at[0, pl.ds(r0, CR), :], p0)
        pltpu.sync_copy(part_hbm.at[1, pl.ds(r0, CR), :], p1)

        def row(i, _):
            for k in range(D // 16):
                p0[i, pl.ds(k * 16, 16)] = (p0[i, pl.ds(k * 16, 16)]
                                            + p1[i, pl.ds(k * 16, 16)])
            return _
        lax.fori_loop(0, CR, row, None)
        pltpu.sync_copy(p0, out_hbm.at[pl.ds(r0, CR), :])
        return _
    lax.fori_loop(0, rpw // CR, chunk, None)


@jax.jit
def _gcn_sc(x_pad, src, dst):
    mesh = plsc.VectorSubcoreMesh(core_axis_name="c", subcore_axis_name="s")
    ka = pl.kernel(
        _main_body,
        out_type=(
            jax.ShapeDtypeStruct((2, NP, D), jnp.float32),
            jax.ShapeDtypeStruct((2 * NP, D), jnp.float32),
        ),
        mesh=mesh,
        compiler_params=pltpu.CompilerParams(needs_layout_passes=False),
        scratch_types=[
            pltpu.VMEM_SHARED((HR, D), jnp.float32),
            pltpu.VMEM_SHARED((NP, D), jnp.float32),
            pltpu.VMEM((HRT, D), jnp.float32),
            pltpu.VMEM((HR,), jnp.int32),
            pltpu.VMEM((HRT, D), jnp.float32),
            pltpu.VMEM((RPT,), jnp.float32),
            pltpu.VMEM((CR, D), jnp.float32),
            pltpu.VMEM((CE,), jnp.int32),
            pltpu.VMEM((CE,), jnp.int32),
            pltpu.VMEM((CE, D), jnp.float32),
            pltpu.VMEM((CR, D), jnp.float32),
            pltpu.SemaphoreType.DMA,
        ],
    )
    part, _ = ka(x_pad, src, dst)
    kb = pl.kernel(
        _sum_body,
        out_type=jax.ShapeDtypeStruct((NP, D), jnp.float32),
        mesh=mesh,
        compiler_params=pltpu.CompilerParams(needs_layout_passes=False),
        scratch_types=[
            pltpu.VMEM((CR, D), jnp.float32),
            pltpu.VMEM((CR, D), jnp.float32),
        ],
    )
    return kb(part)


def kernel(x, edge_index):
    src = edge_index[0].astype(jnp.int32)
    dst = edge_index[1].astype(jnp.int32)
    x_pad = jnp.pad(x, ((0, NP - N), (0, 0)))
    out = _gcn_sc(x_pad, src, dst)
    return out[:N]


# speedup vs baseline: 29.2489x; 2.1847x over previous
import jax
import jax.numpy as jnp
from jax import lax
from jax.experimental import pallas as pl
from jax.experimental.pallas import tpu as pltpu
from jax.experimental.pallas import tpu_sc as plsc

N = 10000
E = 320000
D = 128
NP = 10240
RPT = NP // 16
HR = NP // 128
HRT = HR // 16
EPT = E // 16
CE = 80
EPC = E // 2
EPCT = EPC // 16
NCE = EPCT // CE
SB = 2000
CPS = SB // CE
CZ = 32
NCZ = RPT // CZ


def _rsqrt16(d):
    ii = lax.bitcast_convert_type(d, jnp.int32)
    ii = jnp.int32(0x5F3759DF) - (ii >> 1)
    y = lax.bitcast_convert_type(ii, jnp.float32)
    half = jnp.float32(0.5) * d
    y = y * (jnp.float32(1.5) - half * y * y)
    y = y * (jnp.float32(1.5) - half * y * y)
    y = y * (jnp.float32(1.5) - half * y * y)
    return y


def _main_body(x_hbm, src_hbm, dst_hbm, part_hbm, xs_hbm,
               deg_sh, acc_sh, zdeg, iden, dbuf, ibuf, xrow,
               ssrcb, sdstb, sidxa, didxa, sidxb, didxb,
               rowsa, rowsb, abuf, sema, semb):
    hist = rowsa
    zacc = abuf
    c = lax.axis_index("c")
    s = lax.axis_index("s")
    f0 = jnp.float32(0.0)

    def fill_zacc(i, _):
        for k in range(D // 16):
            zacc[i, pl.ds(k * 16, 16)] = jnp.full((16,), f0)
        return _
    lax.fori_loop(0, CZ, fill_zacc, None)

    def fill_zdeg(i, _):
        for k in range(D // 16):
            zdeg[i, pl.ds(k * 16, 16)] = jnp.full((16,), f0)
        return _
    lax.fori_loop(0, HRT, fill_zdeg, None)

    def fill_iden(k, _):
        iden[pl.ds(k * 16, 16)] = lax.iota(jnp.int32, 16) + k * 16
        return _
    lax.fori_loop(0, HR // 16, fill_iden, None)

    def fill_hist(i, _):
        for k in range(D // 16):
            hist[i, pl.ds(k * 16, 16)] = jnp.full((16,), f0)
        return _
    lax.fori_loop(0, HR, fill_hist, None)

    pltpu.sync_copy(zdeg, deg_sh.at[pl.ds(s * HRT, HRT), :])

    def zero_acc(j, _):
        pltpu.sync_copy(zacc, acc_sh.at[pl.ds(s * RPT + j * CZ, CZ), :])
        return _
    lax.fori_loop(0, NCZ, zero_acc, None)
    plsc.subcore_barrier()

    one16 = jnp.full((16,), jnp.float32(1.0))

    def deg_block(q, _):
        pltpu.sync_copy(src_hbm.at[pl.ds(s * EPT + q * SB, SB)], ssrcb)

        def deg_step(g, _):
            n = ssrcb[pl.ds(g * 16, 16)]
            plsc.addupdate_scatter(hist, [n >> 7, n & 127], one16)
            return _
        lax.fori_loop(0, SB // 16, deg_step, None)
        return _
    lax.fori_loop(0, EPT // SB, deg_block, None)
    pltpu.sync_copy(hist, deg_sh.at[iden], add=True)
    plsc.subcore_barrier()

    pltpu.sync_copy(deg_sh.at[pl.ds(s * HRT, HRT), :], dbuf)

    def inv_step(i, _):
        r = i // 8
        k = i % 8
        d = jnp.maximum(dbuf[r, pl.ds(k * 16, 16)], jnp.float32(1.0))
        ibuf[pl.ds(i * 16, 16)] = _rsqrt16(d)
        return _
    lax.fori_loop(0, RPT // 16, inv_step, None)

    def pre_chunk(j, _):
        r0 = s * RPT + j * CZ
        pltpu.sync_copy(x_hbm.at[pl.ds(r0, CZ), :], xrow)

        def pre_row(i, _):
            idxv = jnp.zeros((16,), jnp.int32) + (j * CZ + i)
            sc = plsc.load_gather(ibuf, [idxv])
            for k in range(D // 16):
                xrow[i, pl.ds(k * 16, 16)] = xrow[i, pl.ds(k * 16, 16)] * sc
            return _
        lax.fori_loop(0, CZ, pre_row, None)
        pltpu.sync_copy(xrow, xs_hbm.at[pl.ds(c * NP + r0, CZ), :])
        return _
    lax.fori_loop(0, NCZ, pre_chunk, None)
    plsc.subcore_barrier()

    base = c * NP

    def handle(j, sidxp, didxp, rowsp, semp):
        @pl.when(j % CPS == 0)
        def _():
            e0 = c * EPC + s * EPCT + (j // CPS) * SB
            pltpu.sync_copy(src_hbm.at[pl.ds(e0, SB)], ssrcb)
            pltpu.sync_copy(dst_hbm.at[pl.ds(e0, SB)], sdstb)

        off = (j % CPS) * CE
        for k in range(CE // 16):
            sidxp[pl.ds(k * 16, 16)] = ssrcb[pl.ds(off + k * 16, 16)] + base
            didxp[pl.ds(k * 16, 16)] = sdstb[pl.ds(off + k * 16, 16)]
        pltpu.async_copy(xs_hbm.at[sidxp], rowsp, semp)

    def finish(sidxp, didxp, rowsp, semp):
        pltpu.make_async_copy(xs_hbm.at[sidxp], rowsp, semp).wait()
        pltpu.sync_copy(rowsp, acc_sh.at[didxp], add=True)

    handle(0, sidxa, didxa, rowsa, sema)

    def edge_pair(t, _):
        handle(2 * t + 1, sidxb, didxb, rowsb, semb)
        finish(sidxa, didxa, rowsa, sema)
        handle(2 * t + 2, sidxa, didxa, rowsa, sema)
        finish(sidxb, didxb, rowsb, semb)
        return _
    lax.fori_loop(0, (NCE - 1) // 2, edge_pair, None)
    finish(sidxa, didxa, rowsa, sema)
    plsc.subcore_barrier()

    def out_chunk(j, _):
        r0 = s * RPT + j * CZ
        pltpu.sync_copy(acc_sh.at[pl.ds(r0, CZ), :], abuf)

        def out_row(i, _):
            idxv = jnp.zeros((16,), jnp.int32) + (j * CZ + i)
            sc = plsc.load_gather(ibuf, [idxv])
            for k in range(D // 16):
                abuf[i, pl.ds(k * 16, 16)] = abuf[i, pl.ds(k * 16, 16)] * sc
            return _
        lax.fori_loop(0, CZ, out_row, None)
        pltpu.sync_copy(abuf, part_hbm.at[c, pl.ds(r0, CZ), :])
        return _
    lax.fori_loop(0, NCZ, out_chunk, None)


def _sum_body(part_hbm, out_hbm, p0, p1):
    c = lax.axis_index("c")
    s = lax.axis_index("s")
    wid = s * 2 + c
    rpw = NP // 32
    CR = 64

    def chunk(j, _):
        r0 = wid * rpw + j * CR
        pltpu.sync_copy(part_hbm.at[0, pl.ds(r0, CR), :], p0)
        pltpu.sync_copy(part_hbm.at[1, pl.ds(r0, CR), :], p1)

        def row(i, _):
            for k in range(D // 16):
                p0[i, pl.ds(k * 16, 16)] = (p0[i, pl.ds(k * 16, 16)]
                                            + p1[i, pl.ds(k * 16, 16)])
            return _
        lax.fori_loop(0, CR, row, None)
        pltpu.sync_copy(p0, out_hbm.at[pl.ds(r0, CR), :])
        return _
    lax.fori_loop(0, rpw // CR, chunk, None)


@jax.jit
def _gcn_sc(x_pad, src, dst):
    mesh = plsc.VectorSubcoreMesh(core_axis_name="c", subcore_axis_name="s")
    ka = pl.kernel(
        _main_body,
        out_type=(
            jax.ShapeDtypeStruct((2, NP, D), jnp.float32),
            jax.ShapeDtypeStruct((2 * NP, D), jnp.float32),
        ),
        mesh=mesh,
        compiler_params=pltpu.CompilerParams(needs_layout_passes=False),
        scratch_types=[
            pltpu.VMEM_SHARED((HR, D), jnp.float32),
            pltpu.VMEM_SHARED((NP, D), jnp.float32),
            pltpu.VMEM((HRT, D), jnp.float32),
            pltpu.VMEM((HR,), jnp.int32),
            pltpu.VMEM((HRT, D), jnp.float32),
            pltpu.VMEM((RPT,), jnp.float32),
            pltpu.VMEM((CZ, D), jnp.float32),
            pltpu.VMEM((SB,), jnp.int32),
            pltpu.VMEM((SB,), jnp.int32),
            pltpu.VMEM((CE,), jnp.int32),
            pltpu.VMEM((CE,), jnp.int32),
            pltpu.VMEM((CE,), jnp.int32),
            pltpu.VMEM((CE,), jnp.int32),
            pltpu.VMEM((CE, D), jnp.float32),
            pltpu.VMEM((CE, D), jnp.float32),
            pltpu.VMEM((CZ, D), jnp.float32),
            pltpu.SemaphoreType.DMA,
            pltpu.SemaphoreType.DMA,
        ],
    )
    part, _ = ka(x_pad, src, dst)
    kb = pl.kernel(
        _sum_body,
        out_type=jax.ShapeDtypeStruct((NP, D), jnp.float32),
        mesh=mesh,
        compiler_params=pltpu.CompilerParams(needs_layout_passes=False),
        scratch_types=[
            pltpu.VMEM((64, D), jnp.float32),
            pltpu.VMEM((64, D), jnp.float32),
        ],
    )
    return kb(part)


def kernel(x, edge_index):
    src = edge_index[0].astype(jnp.int32)
    dst = edge_index[1].astype(jnp.int32)
    x_pad = jnp.pad(x, ((0, NP - N), (0, 0)))
    out = _gcn_sc(x_pad, src, dst)
    return out[:N]
